# baseline (speedup 1.0000x reference)
import functools
import jax
import jax.numpy as jnp
from jax import lax
from jax.experimental import pallas as pl
from jax.experimental.pallas import tpu as pltpu
from jax.experimental.pallas import tpu_sc as plsc

NC = 2
NS = 16
L = 16
NW = NC * NS
KSEG = 1024
STRIP = 4096
SPAN = 32
NPLN = 3
NIL = 3
SUB = 8


def _pool(B, C, HW):
  P = B * C
  PPW = P // NW
  NPASS = PPW // NPLN
  NSTRIP = HW // STRIP
  ACC_W = SUB * KSEG
  mesh = plsc.VectorSubcoreMesh(core_axis_name="c", subcore_axis_name="s")

  @functools.partial(
      pl.kernel,
      mesh=mesh,
      out_type=jax.ShapeDtypeStruct((P, KSEG), jnp.float32),
      compiler_params=pltpu.CompilerParams(
          needs_layout_passes=False, use_tc_tiling_on_sc=False
      ),
      scratch_types=[
          pltpu.VMEM((2, STRIP), jnp.int32),
          pltpu.VMEM((2, NPLN, STRIP), jnp.float32),
      ] + [pltpu.VMEM((ACC_W,), jnp.float32) for _ in range(NPLN * NIL)] + [
          pltpu.VMEM((KSEG,), jnp.float32),
          pltpu.SemaphoreType.DMA,
          pltpu.SemaphoreType.DMA,
      ],
  )
  def k(img_hbm, spx_hbm, out_hbm, lbl_v, d_v, *rest):
    accs = rest[:NPLN * NIL]
    row_v = rest[NPLN * NIL]
    sems = rest[NPLN * NIL + 1:]
    wid = lax.axis_index("s") * NC + lax.axis_index("c")
    lane = lax.iota(jnp.int32, L)
    lane_k = jnp.minimum(lane, 15 - lane) * KSEG
    neg_inf = jnp.full((L,), -jnp.inf, jnp.float32)

    def issue(s, slot, p0, b):
      off = s * STRIP
      pltpu.async_copy(
          spx_hbm.at[b, pl.ds(off, STRIP)], lbl_v.at[slot], sems[slot])
      pltpu.async_copy(
          img_hbm.at[pl.ds(p0, NPLN), pl.ds(off, STRIP)], d_v.at[slot],
          sems[slot])

    def wait(slot):
      pltpu.make_async_copy(
          spx_hbm.at[0, pl.ds(0, STRIP)], lbl_v.at[slot], sems[slot]).wait()
      pltpu.make_async_copy(
          img_hbm.at[pl.ds(0, NPLN), pl.ds(0, STRIP)], d_v.at[slot],
          sems[slot]).wait()

    for ps in range(NPASS):
      p0 = wid * PPW + NPLN * ps
      b = p0 // C

      def init_body(j, _):
        o = j * (4 * L)
        for a in accs:
          for u in range(4):
            a[pl.ds(o + u * L, L)] = neg_inf
        return 0

      lax.fori_loop(0, ACC_W // (4 * L), init_body, 0)

      issue(0, 0, p0, b)

      def process(slot):
        def span_body(t, _):
          base = t * (SPAN * L)
          for g in range(SPAN):
            o = base + g * L
            lbl = lbl_v[slot, pl.ds(o, L)]
            idx = lane_k + lbl
            idx_r = lax.rev(idx, (0,))
            eq = idx == idx_r
            for p in range(NPLN):
              acc = accs[p * NIL + (g % NIL)]
              v = d_v[slot, p, pl.ds(o, L)]
              v2 = jnp.where(eq, jnp.maximum(v, lax.rev(v, (0,))), v)
              c = plsc.load_gather(acc, [idx])
              plsc.store_scatter(acc, [idx], jnp.maximum(c, v2))
          return 0

        lax.fori_loop(0, STRIP // (SPAN * L), span_body, 0)

      def strip_body(s2, _):
        s = s2 * 2
        issue(s + 1, 1, p0, b)
        wait(0)
        process(0)

        @pl.when(s2 + 1 < NSTRIP // 2)
        def _():
          issue(s + 2, 0, p0, b)

        wait(1)
        process(1)
        return 0

      lax.fori_loop(0, NSTRIP // 2, strip_body, 0)

      for p in range(NPLN):
        acc_set = accs[p * NIL:(p + 1) * NIL]

        def fin_body(jj, _):
          m = neg_inf
          for a in acc_set:
            for l in range(SUB):
              m = jnp.maximum(m, a[pl.ds(l * KSEG + jj * L, L)])
          row_v[pl.ds(jj * L, L)] = m
          return 0

        lax.fori_loop(0, KSEG // L, fin_body, 0)
        pltpu.sync_copy(row_v, out_hbm.at[p0 + p])

  return k


@jax.jit
def kernel(img, spx):
  B, C, H, W = img.shape
  HW = H * W
  img2 = img.reshape(B * C, HW)
  spx2 = spx.reshape(B, HW).astype(jnp.int32)
  out = _pool(B, C, HW)(img2, spx2)
  return out.reshape(B, C, KSEG)


# speedup vs baseline: 1.0194x; 1.0194x over previous
import functools
import jax
import jax.numpy as jnp
from jax import lax
from jax.experimental import pallas as pl
from jax.experimental.pallas import tpu as pltpu
from jax.experimental.pallas import tpu_sc as plsc

NC = 2
NS = 16
L = 16
NW = NC * NS
KSEG = 1024
STRIP = 4096
SPAN = 8
NPLN = 3
NIL = 3
SUB = 8


def _pool(B, C, HW):
  P = B * C
  PPW = P // NW
  NPASS = PPW // NPLN
  NSTRIP = HW // STRIP
  ACC_W = SUB * KSEG
  mesh = plsc.VectorSubcoreMesh(core_axis_name="c", subcore_axis_name="s")

  @functools.partial(
      pl.kernel,
      mesh=mesh,
      out_type=jax.ShapeDtypeStruct((P, KSEG), jnp.float32),
      compiler_params=pltpu.CompilerParams(
          needs_layout_passes=False, use_tc_tiling_on_sc=False
      ),
      scratch_types=[
          pltpu.VMEM((2, STRIP), jnp.int32),
          pltpu.VMEM((2, NPLN, STRIP), jnp.float32),
      ] + [pltpu.VMEM((ACC_W,), jnp.float32) for _ in range(NPLN * NIL)] + [
          pltpu.VMEM((KSEG,), jnp.float32),
          pltpu.SemaphoreType.DMA,
          pltpu.SemaphoreType.DMA,
      ],
  )
  def k(img_hbm, spx_hbm, out_hbm, lbl_v, d_v, *rest):
    accs = rest[:NPLN * NIL]
    row_v = rest[NPLN * NIL]
    sems = rest[NPLN * NIL + 1:]
    wid = lax.axis_index("s") * NC + lax.axis_index("c")
    lane = lax.iota(jnp.int32, L)
    lane_k = jnp.minimum(lane, 15 - lane) * KSEG
    neg_inf = jnp.full((L,), -jnp.inf, jnp.float32)

    def issue(s, slot, p0, b):
      off = s * STRIP
      pltpu.async_copy(
          spx_hbm.at[b, pl.ds(off, STRIP)], lbl_v.at[slot], sems[slot])
      pltpu.async_copy(
          img_hbm.at[pl.ds(p0, NPLN), pl.ds(off, STRIP)], d_v.at[slot],
          sems[slot])

    def wait(slot):
      pltpu.make_async_copy(
          spx_hbm.at[0, pl.ds(0, STRIP)], lbl_v.at[slot], sems[slot]).wait()
      pltpu.make_async_copy(
          img_hbm.at[pl.ds(0, NPLN), pl.ds(0, STRIP)], d_v.at[slot],
          sems[slot]).wait()

    for ps in range(NPASS):
      p0 = wid * PPW + NPLN * ps
      b = p0 // C

      def init_body(j, _):
        o = j * (4 * L)
        for a in accs:
          for u in range(4):
            a[pl.ds(o + u * L, L)] = neg_inf
        return 0

      lax.fori_loop(0, ACC_W // (4 * L), init_body, 0)

      issue(0, 0, p0, b)

      def process(slot):
        def span_body(t, _):
          base = t * (SPAN * L)
          for g in range(SPAN):
            o = base + g * L
            lbl = lbl_v[slot, pl.ds(o, L)]
            idx = lane_k + lbl
            idx_r = lax.rev(idx, (0,))
            eq = idx == idx_r
            for p in range(NPLN):
              acc = accs[p * NIL + (g % NIL)]
              v = d_v[slot, p, pl.ds(o, L)]
              v2 = jnp.where(eq, jnp.maximum(v, lax.rev(v, (0,))), v)
              c = plsc.load_gather(acc, [idx])
              plsc.store_scatter(acc, [idx], jnp.maximum(c, v2))
          return 0

        lax.fori_loop(0, STRIP // (SPAN * L), span_body, 0)

      def strip_body(s2, _):
        s = s2 * 2
        issue(s + 1, 1, p0, b)
        wait(0)
        process(0)

        @pl.when(s2 + 1 < NSTRIP // 2)
        def _():
          issue(s + 2, 0, p0, b)

        wait(1)
        process(1)
        return 0

      lax.fori_loop(0, NSTRIP // 2, strip_body, 0)

      for p in range(NPLN):
        acc_set = accs[p * NIL:(p + 1) * NIL]

        def fin_body(jj, _):
          m = neg_inf
          for a in acc_set:
            for l in range(SUB):
              m = jnp.maximum(m, a[pl.ds(l * KSEG + jj * L, L)])
          row_v[pl.ds(jj * L, L)] = m
          return 0

        lax.fori_loop(0, KSEG // L, fin_body, 0)
        pltpu.sync_copy(row_v, out_hbm.at[p0 + p])

  return k


@jax.jit
def kernel(img, spx):
  B, C, H, W = img.shape
  HW = H * W
  img2 = img.reshape(B * C, HW)
  spx2 = spx.reshape(B, HW).astype(jnp.int32)
  out = _pool(B, C, HW)(img2, spx2)
  return out.reshape(B, C, KSEG)
